# baseline (speedup 1.0000x reference)
import functools

import jax
import jax.numpy as jnp
from jax import lax
from jax.experimental import pallas as pl
from jax.experimental.pallas import tpu as pltpu
from jax.experimental.pallas import tpu_sc as plsc

D_MODEL = 64

_NC = 2
_NS = 16
_NW = _NC * _NS
_NB = 6
_GA = 3
_SPLITS = ((0, 128), (128, 72))


def _pe_gather(table, idx):
    R, S = idx.shape
    rows_per_w = R // _NW
    mesh = plsc.VectorSubcoreMesh(core_axis_name="c", subcore_axis_name="s")

    @functools.partial(
        pl.kernel,
        mesh=mesh,
        compiler_params=pltpu.CompilerParams(use_tc_tiling_on_sc=False),
        out_type=jax.ShapeDtypeStruct((R, S, D_MODEL), jnp.float32),
        scratch_types=[
            pltpu.VMEM((rows_per_w, S), jnp.int32),
            pltpu.VMEM((_NB, S, D_MODEL), jnp.float32),
            pltpu.SemaphoreType.DMA((_NB,)),
            pltpu.SemaphoreType.DMA((_NB,)),
        ],
    )
    def k(table_hbm, idx_hbm, out_hbm, idx_v, rows_v, gsem, wsem):
        wid = lax.axis_index("s") * _NC + lax.axis_index("c")
        base = wid * rows_per_w
        pltpu.sync_copy(idx_hbm.at[pl.ds(base, rows_per_w)], idx_v)

        def issue_gather(chunk, buf):
            for off, ln in _SPLITS:
                pltpu.async_copy(
                    table_hbm.at[idx_v.at[chunk, pl.ds(off, ln)]],
                    rows_v.at[buf, pl.ds(off, ln)],
                    gsem.at[buf],
                )

        for j in range(_GA):
            issue_gather(j, j)

        def body(i, carry):
            b = lax.rem(i, _NB)
            pltpu.make_async_copy(
                table_hbm.at[pl.ds(0, S)], rows_v.at[b], gsem.at[b]
            ).wait()
            pltpu.async_copy(rows_v.at[b], out_hbm.at[base + i], wsem.at[b])
            nxt = i + _GA

            @pl.when(nxt < rows_per_w)
            def _():
                bn = lax.rem(nxt, _NB)

                @pl.when(nxt >= _NB)
                def _():
                    pltpu.make_async_copy(
                        rows_v.at[bn], out_hbm.at[base], wsem.at[bn]
                    ).wait()

                issue_gather(nxt, bn)

            return carry

        lax.fori_loop(0, rows_per_w, body, 0)

        for j in range(_NB):
            pltpu.make_async_copy(
                rows_v.at[j], out_hbm.at[base], wsem.at[j]
            ).wait()

    return k(table, idx)


def kernel(indices, pe):
    return _pe_gather(pe, indices.astype(jnp.int32))
